# Initial kernel scaffold; baseline (speedup 1.0000x reference)
import jax
import jax.numpy as jnp
from jax.experimental import pallas as pl


def kernel(z, codebook):
    raise NotImplementedError("write your pallas kernel here")


# speedup vs baseline: 1.1663x; 1.1663x over previous
import functools

import jax
import jax.numpy as jnp
from jax import lax
from jax.experimental import pallas as pl
from jax.experimental.pallas import tpu as pltpu
from jax.experimental.pallas import tpu_sc as plsc

N_ROWS = 16384
D = 64
N_CODES = 8192

BZ = 256
BC = 2048
NB = N_ROWS // BZ
NCHUNK = N_CODES // BC


def _tc_body(z_ref, cb_ref, cn_ref, idx_ref, loss_ref):
    i = pl.program_id(0)
    zb = z_ref[...]
    znorm = jnp.sum(zb * zb, axis=1, keepdims=True)

    def chunk_min(j):
        cb = cb_ref[pl.ds(j * BC, BC), :]
        cn = cn_ref[:, pl.ds(j * BC, BC)]
        scores = lax.dot_general(zb, cb, (((1,), (1,)), ((), ())),
                                 preferred_element_type=jnp.float32)
        d = (znorm + cn) - 2.0 * scores
        m = jnp.min(d, axis=1, keepdims=True)
        iot = lax.broadcasted_iota(jnp.int32, (BZ, BC), 1) + j * BC
        a = jnp.min(jnp.where(d == m, iot, jnp.int32(2**30)),
                    axis=1, keepdims=True)
        return m, a

    def merge(mv_a, mv_b):
        (m_a, a_a), (m_b, a_b) = mv_a, mv_b
        upd = m_b < m_a
        return jnp.where(upd, m_b, m_a), jnp.where(upd, a_b, a_a)

    half1 = merge(chunk_min(0), chunk_min(1))
    half2 = merge(chunk_min(2), chunk_min(3))

    m1b = half1[0].astype(jnp.bfloat16).astype(jnp.float32)
    take2 = half2[0] < m1b
    best_idx = jnp.where(take2, half2[1], half1[1])
    best_val = jnp.where(take2, half2[0], half1[0])

    idx_ref[...] = best_idx

    @pl.when(i == 0)
    def _init():
        loss_ref[...] = jnp.zeros_like(loss_ref)

    loss_ref[...] += jnp.sum(best_val, keepdims=True).reshape(1, 1)


def _tc_stage(z, codebook, cnorm):
    return pl.pallas_call(
        _tc_body,
        grid=(NB,),
        in_specs=[
            pl.BlockSpec((BZ, D), lambda i: (i, 0)),
            pl.BlockSpec((N_CODES, D), lambda i: (0, 0)),
            pl.BlockSpec((1, N_CODES), lambda i: (0, 0)),
        ],
        out_specs=[
            pl.BlockSpec((BZ, 1), lambda i: (i, 0)),
            pl.BlockSpec((1, 1), lambda i: (0, 0)),
        ],
        out_shape=[
            jax.ShapeDtypeStruct((N_ROWS, 1), jnp.int32),
            jax.ShapeDtypeStruct((1, 1), jnp.float32),
        ],
    )(z, codebook, cnorm)


_NC, _NS = 2, 16
_NW = _NC * _NS
_BPW = N_ROWS // _NW
_IDX_CHUNK = 128
_NCH = _BPW // _IDX_CHUNK


def _sc_gather_body(cb_hbm, idx_hbm, out_hbm, idx_v, rows_v, sem):
    wid = lax.axis_index("s") * _NC + lax.axis_index("c")
    base = wid * _BPW
    pltpu.sync_copy(idx_hbm.at[pl.ds(wid * _NCH, _NCH), :], idx_v)
    copies = [
        pltpu.async_copy(cb_hbm.at[idx_v.at[j]],
                         rows_v.at[pl.ds(j * _IDX_CHUNK, _IDX_CHUNK)], sem)
        for j in range(_NCH)
    ]
    for cp in copies:
        cp.wait()
    pltpu.sync_copy(rows_v, out_hbm.at[pl.ds(base, _BPW), :])


@functools.lru_cache(maxsize=1)
def _sc_gather():
    return pl.kernel(
        _sc_gather_body,
        mesh=plsc.VectorSubcoreMesh(core_axis_name="c", subcore_axis_name="s"),
        compiler_params=pltpu.CompilerParams(use_tc_tiling_on_sc=False),
        out_type=jax.ShapeDtypeStruct((N_ROWS, D), jnp.float32),
        scratch_types=[
            pltpu.VMEM((_NCH, _IDX_CHUNK), jnp.int32),
            pltpu.VMEM((_BPW, D), jnp.float32),
            pltpu.SemaphoreType.DMA,
        ],
    )


def kernel(z, codebook):
    cnorm = jnp.sum(codebook ** 2, axis=-1).reshape(1, N_CODES)
    idx2d, loss_sum = _tc_stage(z, codebook, cnorm)
    idx = idx2d.reshape(N_ROWS)
    z_q = _sc_gather()(codebook, idx2d.reshape(N_ROWS // _IDX_CHUNK, _IDX_CHUNK))
    loss = 0.25 * loss_sum[0, 0] / (N_ROWS * D)
    return (z_q, loss, idx)


# speedup vs baseline: 1.2627x; 1.0827x over previous
import functools

import jax
import jax.numpy as jnp
from jax import lax
from jax.experimental import pallas as pl
from jax.experimental.pallas import tpu as pltpu
from jax.experimental.pallas import tpu_sc as plsc

N_ROWS = 16384
D = 64
N_CODES = 8192

BZ = 512
BC = 2048
NB = N_ROWS // BZ
NCHUNK = N_CODES // BC


def _tc_body(z_ref, cb_ref, cn_ref, idx_ref, loss_ref):
    i = pl.program_id(0)
    zb = z_ref[...]
    znorm = jnp.sum(zb * zb, axis=1, keepdims=True)

    iot = lax.broadcasted_iota(jnp.int32, (BZ, BC), 1)

    def chunk_min(j):
        cb = cb_ref[pl.ds(j * BC, BC), :]
        cn = cn_ref[:, pl.ds(j * BC, BC)]
        scores = lax.dot_general(zb, cb, (((1,), (1,)), ((), ())),
                                 preferred_element_type=jnp.float32)
        d = (znorm + cn) - 2.0 * scores
        m = jnp.min(d, axis=1, keepdims=True)
        a = jnp.min(jnp.where(d == m, iot, jnp.int32(BC)),
                    axis=1, keepdims=True) + j * BC
        return m, a

    def merge(mv_a, mv_b):
        (m_a, a_a), (m_b, a_b) = mv_a, mv_b
        upd = m_b < m_a
        return jnp.where(upd, m_b, m_a), jnp.where(upd, a_b, a_a)

    half1 = merge(chunk_min(0), chunk_min(1))
    half2 = merge(chunk_min(2), chunk_min(3))

    m1b = half1[0].astype(jnp.bfloat16).astype(jnp.float32)
    take2 = half2[0] < m1b
    best_idx = jnp.where(take2, half2[1], half1[1])
    best_val = jnp.where(take2, half2[0], half1[0])

    idx_ref[...] = best_idx

    @pl.when(i == 0)
    def _init():
        loss_ref[...] = jnp.zeros_like(loss_ref)

    loss_ref[...] += jnp.sum(best_val, keepdims=True).reshape(1, 1)


def _tc_stage(z, codebook, cnorm):
    return pl.pallas_call(
        _tc_body,
        grid=(NB,),
        in_specs=[
            pl.BlockSpec((BZ, D), lambda i: (i, 0)),
            pl.BlockSpec((N_CODES, D), lambda i: (0, 0)),
            pl.BlockSpec((1, N_CODES), lambda i: (0, 0)),
        ],
        out_specs=[
            pl.BlockSpec((BZ, 1), lambda i: (i, 0)),
            pl.BlockSpec((1, 1), lambda i: (0, 0)),
        ],
        out_shape=[
            jax.ShapeDtypeStruct((N_ROWS, 1), jnp.int32),
            jax.ShapeDtypeStruct((1, 1), jnp.float32),
        ],
    )(z, codebook, cnorm)


_NC, _NS = 2, 16
_NW = _NC * _NS
_BPW = N_ROWS // _NW
_IDX_CHUNK = 128
_NCH = _BPW // _IDX_CHUNK


def _sc_gather_body(cb_hbm, idx_hbm, out_hbm, idx_v, rows_v, sem):
    wid = lax.axis_index("s") * _NC + lax.axis_index("c")
    base = wid * _BPW
    pltpu.sync_copy(idx_hbm.at[pl.ds(wid * _NCH, _NCH), :], idx_v)
    copies = [
        pltpu.async_copy(cb_hbm.at[idx_v.at[j]],
                         rows_v.at[pl.ds(j * _IDX_CHUNK, _IDX_CHUNK)], sem)
        for j in range(_NCH)
    ]
    for cp in copies:
        cp.wait()
    pltpu.sync_copy(rows_v, out_hbm.at[pl.ds(base, _BPW), :])


@functools.lru_cache(maxsize=1)
def _sc_gather():
    return pl.kernel(
        _sc_gather_body,
        mesh=plsc.VectorSubcoreMesh(core_axis_name="c", subcore_axis_name="s"),
        compiler_params=pltpu.CompilerParams(use_tc_tiling_on_sc=False),
        out_type=jax.ShapeDtypeStruct((N_ROWS, D), jnp.float32),
        scratch_types=[
            pltpu.VMEM((_NCH, _IDX_CHUNK), jnp.int32),
            pltpu.VMEM((_BPW, D), jnp.float32),
            pltpu.SemaphoreType.DMA,
        ],
    )


def kernel(z, codebook):
    cnorm = jnp.sum(codebook ** 2, axis=-1).reshape(1, N_CODES)
    idx2d, loss_sum = _tc_stage(z, codebook, cnorm)
    idx = idx2d.reshape(N_ROWS)
    z_q = _sc_gather()(codebook, idx2d.reshape(N_ROWS // _IDX_CHUNK, _IDX_CHUNK))
    loss = 0.25 * loss_sum[0, 0] / (N_ROWS * D)
    return (z_q, loss, idx)


# speedup vs baseline: 1.3169x; 1.0429x over previous
import functools

import jax
import jax.numpy as jnp
from jax import lax
from jax.experimental import pallas as pl
from jax.experimental.pallas import tpu as pltpu
from jax.experimental.pallas import tpu_sc as plsc

N_ROWS = 16384
D = 64
N_CODES = 8192

BZ = 1024
BC = 2048
NB = N_ROWS // BZ
NCHUNK = N_CODES // BC


def _tc_body(z_ref, cb_ref, cn_ref, idx_ref, loss_ref):
    i = pl.program_id(0)
    zb = z_ref[...]
    znorm = jnp.sum(zb * zb, axis=1, keepdims=True)

    iot = lax.broadcasted_iota(jnp.int32, (BZ, BC), 1)

    def chunk_min(j):
        cb = cb_ref[pl.ds(j * BC, BC), :]
        cn = cn_ref[:, pl.ds(j * BC, BC)]
        scores = lax.dot_general(zb, cb, (((1,), (1,)), ((), ())),
                                 preferred_element_type=jnp.float32)
        d = (znorm + cn) - 2.0 * scores
        m = jnp.min(d, axis=1, keepdims=True)
        a = jnp.min(jnp.where(d == m, iot, jnp.int32(BC)),
                    axis=1, keepdims=True) + j * BC
        return m, a

    def merge(mv_a, mv_b):
        (m_a, a_a), (m_b, a_b) = mv_a, mv_b
        upd = m_b < m_a
        return jnp.where(upd, m_b, m_a), jnp.where(upd, a_b, a_a)

    half1 = merge(chunk_min(0), chunk_min(1))
    half2 = merge(chunk_min(2), chunk_min(3))

    m1b = half1[0].astype(jnp.bfloat16).astype(jnp.float32)
    take2 = half2[0] < m1b
    best_idx = jnp.where(take2, half2[1], half1[1])
    best_val = jnp.where(take2, half2[0], half1[0])

    idx_ref[...] = best_idx

    @pl.when(i == 0)
    def _init():
        loss_ref[...] = jnp.zeros_like(loss_ref)

    loss_ref[...] += jnp.sum(best_val, keepdims=True).reshape(1, 1)


def _tc_stage(z, codebook, cnorm):
    return pl.pallas_call(
        _tc_body,
        grid=(NB,),
        in_specs=[
            pl.BlockSpec((BZ, D), lambda i: (i, 0)),
            pl.BlockSpec((N_CODES, D), lambda i: (0, 0)),
            pl.BlockSpec((1, N_CODES), lambda i: (0, 0)),
        ],
        out_specs=[
            pl.BlockSpec((BZ, 1), lambda i: (i, 0)),
            pl.BlockSpec((1, 1), lambda i: (0, 0)),
        ],
        out_shape=[
            jax.ShapeDtypeStruct((N_ROWS, 1), jnp.int32),
            jax.ShapeDtypeStruct((1, 1), jnp.float32),
        ],
    )(z, codebook, cnorm)


_NC, _NS = 2, 16
_NW = _NC * _NS
_BPW = N_ROWS // _NW
_IDX_CHUNK = 128
_NCH = _BPW // _IDX_CHUNK


def _sc_gather_body(cb_hbm, idx_hbm, out_hbm, idx_v, rows_v, sem):
    wid = lax.axis_index("s") * _NC + lax.axis_index("c")
    base = wid * _BPW
    pltpu.sync_copy(idx_hbm.at[pl.ds(wid * _NCH, _NCH), :], idx_v)
    copies = [
        pltpu.async_copy(cb_hbm.at[idx_v.at[j]],
                         rows_v.at[pl.ds(j * _IDX_CHUNK, _IDX_CHUNK)], sem)
        for j in range(_NCH)
    ]
    for cp in copies:
        cp.wait()
    pltpu.sync_copy(rows_v, out_hbm.at[pl.ds(base, _BPW), :])


@functools.lru_cache(maxsize=1)
def _sc_gather():
    return pl.kernel(
        _sc_gather_body,
        mesh=plsc.VectorSubcoreMesh(core_axis_name="c", subcore_axis_name="s"),
        compiler_params=pltpu.CompilerParams(use_tc_tiling_on_sc=False),
        out_type=jax.ShapeDtypeStruct((N_ROWS, D), jnp.float32),
        scratch_types=[
            pltpu.VMEM((_NCH, _IDX_CHUNK), jnp.int32),
            pltpu.VMEM((_BPW, D), jnp.float32),
            pltpu.SemaphoreType.DMA,
        ],
    )


def kernel(z, codebook):
    cnorm = jnp.sum(codebook ** 2, axis=-1).reshape(1, N_CODES)
    idx2d, loss_sum = _tc_stage(z, codebook, cnorm)
    idx = idx2d.reshape(N_ROWS)
    z_q = _sc_gather()(codebook, idx2d.reshape(N_ROWS // _IDX_CHUNK, _IDX_CHUNK))
    loss = 0.25 * loss_sum[0, 0] / (N_ROWS * D)
    return (z_q, loss, idx)
